# baseline (speedup 1.0000x reference)
import jax
import jax.numpy as jnp
from jax import lax
from jax.experimental import pallas as pl
from jax.experimental.pallas import tpu as pltpu
from jax.experimental.pallas import tpu_sc as plsc

T, C, H, E, K = 2048, 768, 2048, 8, 2
G = 256
N_TILES = (T * K + E * (G - 1) + G - 1) // G
T_PAD = N_TILES * G
TG = 256
L = 16

_SC_MESH = dict(core_axis_name="c", subcore_axis_name="s", num_cores=2,
                num_subcores=16)


def _router_body(x_ref, gw_ref, logits_ref, i0_ref, i1_ref, w0_ref, w1o_ref):
    x = x_ref[...]
    logits = lax.dot_general(x, gw_ref[...], (((1,), (1,)), ((), ())),
                             preferred_element_type=jnp.float32)
    logits_ref[...] = logits
    idx = lax.broadcasted_iota(jnp.int32, logits.shape, 1)
    m1 = jnp.max(logits, axis=1, keepdims=True)
    i0 = jnp.min(jnp.where(logits == m1, idx, E), axis=1, keepdims=True)
    masked = jnp.where(idx == i0, -jnp.inf, logits)
    m2 = jnp.max(masked, axis=1, keepdims=True)
    i1 = jnp.min(jnp.where(masked == m2, idx, E), axis=1, keepdims=True)
    d = jnp.exp(m2 - m1)
    i0_ref[...] = i0
    i1_ref[...] = i1
    w0_ref[...] = 1.0 / (1.0 + d)
    w1o_ref[...] = d / (1.0 + d)


def _router(x, gate_w):
    return pl.pallas_call(
        _router_body,
        out_shape=[
            jax.ShapeDtypeStruct((T, E), jnp.float32),
            jax.ShapeDtypeStruct((T, 1), jnp.int32),
            jax.ShapeDtypeStruct((T, 1), jnp.int32),
            jax.ShapeDtypeStruct((T, 1), jnp.float32),
            jax.ShapeDtypeStruct((T, 1), jnp.float32),
        ],
    )(x, gate_w)


def _shared_body(x_ref, w1_ref, w3_ref, w2_ref, sh_ref):
    x = x_ref[...]
    h = lax.dot_general(x, w1_ref[...], (((1,), (1,)), ((), ())),
                        preferred_element_type=jnp.float32)
    g = lax.dot_general(x, w3_ref[...], (((1,), (1,)), ((), ())),
                        preferred_element_type=jnp.float32)
    u = (h * jax.nn.sigmoid(h)) * g
    sh_ref[...] = lax.dot_general(u, w2_ref[...], (((1,), (1,)), ((), ())),
                                  preferred_element_type=jnp.float32)


def _shared(x, sw1, sw3, sw2):
    n = T // TG
    return pl.pallas_call(
        _shared_body,
        grid=(n,),
        in_specs=[
            pl.BlockSpec((TG, C), lambda i: (i, 0)),
            pl.BlockSpec((H, C), lambda i: (0, 0)),
            pl.BlockSpec((H, C), lambda i: (0, 0)),
            pl.BlockSpec((C, H), lambda i: (0, 0)),
        ],
        out_specs=pl.BlockSpec((TG, C), lambda i: (i, 0)),
        out_shape=jax.ShapeDtypeStruct((T, C), jnp.float32),
    )(x, sw1, sw3, sw2)


def _dispatch_body(i0_hbm, i1_hbm, w0_hbm, w1_hbm,
                   gidx_hbm, slot0_hbm, slot1_hbm, wslot_hbm, te_hbm,
                   ev_ref, wv_ref, slots_ref, gidx_ref, wsl_ref, te_ref, sem):
    wid = lax.axis_index("s") * 2 + lax.axis_index("c")

    @pl.when(wid == 0)
    def _():
        pltpu.sync_copy(i0_hbm, ev_ref.at[pl.ds(0, T)])
        pltpu.sync_copy(i1_hbm, ev_ref.at[pl.ds(T, T)])
        pltpu.sync_copy(w0_hbm, wv_ref.at[pl.ds(0, T)])
        pltpu.sync_copy(w1_hbm, wv_ref.at[pl.ds(T, T)])

        nv = (2 * T) // L

        def c_body(i, cnts):
            v = ev_ref[pl.ds(i * L, L)]
            return tuple(cnts[e] + jnp.sum(jnp.where(v == e, 1, 0))
                         for e in range(E))

        cnts = lax.fori_loop(0, nv, c_body, (jnp.int32(0),) * E)

        base, nt_cum = [], []
        off = jnp.int32(0)
        tcum = jnp.int32(0)
        for e in range(E):
            p = ((cnts[e] + G - 1) // G) * G
            base.append(off)
            off = off + p
            tcum = tcum + p // G
            nt_cum.append(tcum)

        def z_body(i, _):
            lane = lax.iota(jnp.int32, L)
            gidx_ref[pl.ds(i * L, L)] = lax.bitwise_and(i * L + lane, T - 1)
            wsl_ref[pl.ds(i * L, L)] = jnp.zeros((L,), jnp.float32)
            return 0

        lax.fori_loop(0, T_PAD // L, z_body, 0)

        def p2_body(i, cnts):
            v = ev_ref[pl.ds(i * L, L)]
            w = wv_ref[pl.ds(i * L, L)]
            lane = lax.iota(jnp.int32, L)
            tok = lax.bitwise_and(i * L + lane, T - 1)
            slot_v = jnp.zeros((L,), jnp.int32)
            new = []
            for e in range(E):
                m = v == e
                cs = plsc.cumsum(jnp.where(m, 1, 0))
                slot_v = jnp.where(m, base[e] + cnts[e] + cs - 1, slot_v)
                new.append(cnts[e] + cs[L - 1])
            slots_ref[pl.ds(i * L, L)] = slot_v
            plsc.store_scatter(gidx_ref, [slot_v], tok)
            plsc.store_scatter(wsl_ref, [slot_v], w)
            return tuple(new)

        lax.fori_loop(0, nv, p2_body, (jnp.int32(0),) * E)

        for r in range(2):
            lane = lax.iota(jnp.int32, L) + r * L
            tev = jnp.zeros((L,), jnp.int32)
            for e in range(E):
                tev = tev + jnp.where(lane >= nt_cum[e], 1, 0)
            te_ref[pl.ds(r * L, L)] = jnp.minimum(tev, E - 1)

        pltpu.sync_copy(slots_ref.at[pl.ds(0, T)], slot0_hbm)
        pltpu.sync_copy(slots_ref.at[pl.ds(T, T)], slot1_hbm)
        pltpu.sync_copy(gidx_ref, gidx_hbm)
        pltpu.sync_copy(wsl_ref, wslot_hbm)
        pltpu.sync_copy(te_ref, te_hbm)


def _dispatch(i0, i1, w0, w1):
    f = pl.kernel(
        _dispatch_body,
        out_type=[
            jax.ShapeDtypeStruct((T_PAD,), jnp.int32),
            jax.ShapeDtypeStruct((T,), jnp.int32),
            jax.ShapeDtypeStruct((T,), jnp.int32),
            jax.ShapeDtypeStruct((T_PAD,), jnp.float32),
            jax.ShapeDtypeStruct((2 * L,), jnp.int32),
        ],
        mesh=plsc.VectorSubcoreMesh(**_SC_MESH),
        scratch_types=[
            pltpu.VMEM((2 * T,), jnp.int32),
            pltpu.VMEM((2 * T,), jnp.float32),
            pltpu.VMEM((2 * T,), jnp.int32),
            pltpu.VMEM((T_PAD,), jnp.int32),
            pltpu.VMEM((T_PAD,), jnp.float32),
            pltpu.VMEM((2 * L,), jnp.int32),
            pltpu.SemaphoreType.DMA,
        ],
        compiler_params=pltpu.CompilerParams(needs_layout_passes=False),
    )
    return f(i0, i1, w0, w1)


_G_ROWS = T_PAD // 32
_G_CHUNK = 24
_G_N = _G_ROWS // _G_CHUNK
_G_STAGE = T // 16


def _gather_body(x_hbm, gidx_hbm, xg_hbm,
                 idx_ref, b0, b1, b2, b3, s0, s1, s2, s3, w0, w1, w2, w3):
    cid = lax.axis_index("c")
    sid = lax.axis_index("s")
    wid = sid * 2 + cid
    base = wid * _G_ROWS
    pltpu.sync_copy(gidx_hbm.at[pl.ds(base, _G_ROWS)], idx_ref)
    bufs = (b0, b1, b2, b3)
    sems = (s0, s1, s2, s3)
    wsems = (w0, w1, w2, w3)
    pend_g, pend_w = {}, {}
    for c in range(_G_N):
        if c >= 4:
            pend_w.pop(c - 4).wait()
        pend_g[c] = pltpu.async_copy(
            x_hbm.at[idx_ref.at[pl.ds(c * _G_CHUNK, _G_CHUNK)]],
            bufs[c % 4], sems[c % 4])
        if c >= 3:
            d = c - 3
            pend_g.pop(d).wait()
            pend_w[d] = pltpu.async_copy(
                bufs[d % 4], xg_hbm.at[pl.ds(base + d * _G_CHUNK, _G_CHUNK)],
                wsems[d % 4])
    for d in range(_G_N - 3, _G_N):
        pend_g.pop(d).wait()
        pend_w[d] = pltpu.async_copy(
            bufs[d % 4], xg_hbm.at[pl.ds(base + d * _G_CHUNK, _G_CHUNK)],
            wsems[d % 4])
    for d in pend_w:
        pend_w[d].wait()


def _gather(x2d, gidx):
    f = pl.kernel(
        _gather_body,
        out_type=jax.ShapeDtypeStruct((T_PAD, C), jnp.float32),
        mesh=plsc.VectorSubcoreMesh(**_SC_MESH),
        scratch_types=[
            pltpu.VMEM((_G_ROWS,), jnp.int32),
            pltpu.VMEM((_G_CHUNK, C), jnp.float32),
            pltpu.VMEM((_G_CHUNK, C), jnp.float32),
            pltpu.VMEM((_G_CHUNK, C), jnp.float32),
            pltpu.VMEM((_G_CHUNK, C), jnp.float32),
            pltpu.SemaphoreType.DMA,
            pltpu.SemaphoreType.DMA,
            pltpu.SemaphoreType.DMA,
            pltpu.SemaphoreType.DMA,
            pltpu.SemaphoreType.DMA,
            pltpu.SemaphoreType.DMA,
            pltpu.SemaphoreType.DMA,
            pltpu.SemaphoreType.DMA,
        ],
    )
    return f(x2d, gidx)


def _grouped_body(te_ref, xg_ref, w1_ref, w3_ref, w2_ref, ws_ref, out_ref):
    xg = xg_ref[...]
    h = lax.dot_general(xg, w1_ref[0], (((1,), (1,)), ((), ())),
                        preferred_element_type=jnp.float32)
    g = lax.dot_general(xg, w3_ref[0], (((1,), (1,)), ((), ())),
                        preferred_element_type=jnp.float32)
    u = (h * jax.nn.sigmoid(h)) * g
    y = lax.dot_general(u, w2_ref[0], (((1,), (1,)), ((), ())),
                        preferred_element_type=jnp.float32)
    out_ref[...] = y * ws_ref[...]


def _grouped(xg, ew1, ew3, ew2, wslot2d, tile_expert):
    grid_spec = pltpu.PrefetchScalarGridSpec(
        num_scalar_prefetch=1,
        grid=(N_TILES,),
        in_specs=[
            pl.BlockSpec((G, C), lambda i, te: (i, 0)),
            pl.BlockSpec((1, H, C), lambda i, te: (te[i], 0, 0)),
            pl.BlockSpec((1, H, C), lambda i, te: (te[i], 0, 0)),
            pl.BlockSpec((1, C, H), lambda i, te: (te[i], 0, 0)),
            pl.BlockSpec((G, 1), lambda i, te: (i, 0)),
        ],
        out_specs=pl.BlockSpec((G, C), lambda i, te: (i, 0)),
    )
    return pl.pallas_call(
        _grouped_body,
        grid_spec=grid_spec,
        out_shape=jax.ShapeDtypeStruct((T_PAD, C), jnp.float32),
    )(tile_expert, xg, ew1, ew3, ew2, wslot2d)


_C_TOK = 64
_C_CHUNK = 8
_C_N = _C_TOK // _C_CHUNK


def _combine_body(sh_hbm, yg_hbm, slot0_hbm, slot1_hbm, out_hbm,
                  idx0_ref, idx1_ref, *rest):
    accs = rest[0:4]
    r0s = rest[4:10]
    r1s = rest[10:16]
    shsems = rest[16:20]
    osems = rest[20:24]
    sems = rest[24:30]
    wid = lax.axis_index("s") * 2 + lax.axis_index("c")
    base = wid * _C_TOK
    pltpu.sync_copy(slot0_hbm.at[pl.ds(base, _C_TOK)], idx0_ref)
    pltpu.sync_copy(slot1_hbm.at[pl.ds(base, _C_TOK)], idx1_ref)
    pending = {}
    pend_w = {}

    def issue(c):
        s = pl.ds(c * _C_CHUNK, _C_CHUNK)
        a = c % 4
        if c >= 4:
            pend_w.pop(c - 4).wait()
        h0 = pltpu.async_copy(yg_hbm.at[idx0_ref.at[s]], r0s[c % 6],
                              sems[c % 6])
        h1 = pltpu.async_copy(yg_hbm.at[idx1_ref.at[s]], r1s[c % 6],
                              sems[c % 6])
        hs = pltpu.async_copy(sh_hbm.at[pl.ds(base + c * _C_CHUNK, _C_CHUNK)],
                              accs[a], shsems[a])
        pending[c] = (h0, h1, hs)

    def finish(c):
        b = c % 6
        a = c % 4
        h0, h1, hs = pending.pop(c)
        o = base + c * _C_CHUNK
        hs.wait()
        h0.wait()
        h1.wait()

        def add_body(j, _):
            t = j // (C // L)
            k = (j % (C // L)) * L
            accs[a][t, pl.ds(k, L)] = (accs[a][t, pl.ds(k, L)]
                                       + r0s[b][t, pl.ds(k, L)]
                                       + r1s[b][t, pl.ds(k, L)])
            return 0

        lax.fori_loop(0, _C_CHUNK * (C // L), add_body, 0, unroll=4)
        pend_w[c] = pltpu.async_copy(accs[a], out_hbm.at[pl.ds(o, _C_CHUNK)],
                                     osems[a])

    for c in range(_C_N):
        if c >= 3:
            finish(c - 3)
        issue(c)
    for d in range(_C_N - 3, _C_N):
        finish(d)
    for d in pend_w:
        pend_w[d].wait()


def _combine(sh, yg, slot0, slot1):
    f = pl.kernel(
        _combine_body,
        out_type=jax.ShapeDtypeStruct((T, C), jnp.float32),
        mesh=plsc.VectorSubcoreMesh(**_SC_MESH),
        scratch_types=(
            [pltpu.VMEM((_C_TOK,), jnp.int32)] * 2
            + [pltpu.VMEM((_C_CHUNK, C), jnp.float32)] * 16
            + [pltpu.SemaphoreType.DMA] * 14
        ),
    )
    return f(sh, yg, slot0, slot1)


def kernel(x, gate_w, sw1, sw3, sw2, ew1, ew3, ew2):
    x2d = x.reshape(T, C)
    logits, i0, i1, w0, w1 = _router(x2d, gate_w)
    gidx, slot0, slot1, wslot, te = _dispatch(
        i0.reshape(T), i1.reshape(T), w0.reshape(T), w1.reshape(T))
    sh = _shared(x2d, sw1, sw3, sw2)
    xg = _gather(x2d, gidx)
    yg = _grouped(xg, ew1, ew3, ew2, wslot.reshape(T_PAD, 1), te)
    out = _combine(sh, yg, slot0, slot1)
    return out.reshape(x.shape), logits.reshape(x.shape[0], T, E)


# speedup vs baseline: 1.0423x; 1.0423x over previous
import jax
import jax.numpy as jnp
from jax import lax
from jax.experimental import pallas as pl
from jax.experimental.pallas import tpu as pltpu
from jax.experimental.pallas import tpu_sc as plsc

T, C, H, E, K = 2048, 768, 2048, 8, 2
G = 256
N_TILES = (T * K + E * (G - 1) + G - 1) // G
T_PAD = N_TILES * G
TG = 256
L = 16

_SC_MESH = dict(core_axis_name="c", subcore_axis_name="s", num_cores=2,
                num_subcores=16)


def _router_body(x_ref, gw_ref, logits_ref, i0_ref, i1_ref, w0_ref, w1o_ref):
    x = x_ref[...]
    logits = lax.dot_general(x, gw_ref[...], (((1,), (1,)), ((), ())),
                             preferred_element_type=jnp.float32)
    logits_ref[...] = logits
    idx = lax.broadcasted_iota(jnp.int32, logits.shape, 1)
    m1 = jnp.max(logits, axis=1, keepdims=True)
    i0 = jnp.min(jnp.where(logits == m1, idx, E), axis=1, keepdims=True)
    masked = jnp.where(idx == i0, -jnp.inf, logits)
    m2 = jnp.max(masked, axis=1, keepdims=True)
    i1 = jnp.min(jnp.where(masked == m2, idx, E), axis=1, keepdims=True)
    d = jnp.exp(m2 - m1)
    i0_ref[...] = i0
    i1_ref[...] = i1
    w0_ref[...] = 1.0 / (1.0 + d)
    w1o_ref[...] = d / (1.0 + d)


def _router(x, gate_w):
    return pl.pallas_call(
        _router_body,
        out_shape=[
            jax.ShapeDtypeStruct((T, E), jnp.float32),
            jax.ShapeDtypeStruct((T, 1), jnp.int32),
            jax.ShapeDtypeStruct((T, 1), jnp.int32),
            jax.ShapeDtypeStruct((T, 1), jnp.float32),
            jax.ShapeDtypeStruct((T, 1), jnp.float32),
        ],
    )(x, gate_w)


def _shared_body(x_ref, w1_ref, w3_ref, w2_ref, sh_ref):
    x = x_ref[...]
    h = lax.dot_general(x, w1_ref[...], (((1,), (1,)), ((), ())),
                        preferred_element_type=jnp.float32)
    g = lax.dot_general(x, w3_ref[...], (((1,), (1,)), ((), ())),
                        preferred_element_type=jnp.float32)
    u = (h * jax.nn.sigmoid(h)) * g
    sh_ref[...] = lax.dot_general(u, w2_ref[...], (((1,), (1,)), ((), ())),
                                  preferred_element_type=jnp.float32)


def _shared(x, sw1, sw3, sw2):
    n = T // TG
    return pl.pallas_call(
        _shared_body,
        grid=(n,),
        in_specs=[
            pl.BlockSpec((TG, C), lambda i: (i, 0)),
            pl.BlockSpec((H, C), lambda i: (0, 0)),
            pl.BlockSpec((H, C), lambda i: (0, 0)),
            pl.BlockSpec((C, H), lambda i: (0, 0)),
        ],
        out_specs=pl.BlockSpec((TG, C), lambda i: (i, 0)),
        out_shape=jax.ShapeDtypeStruct((T, C), jnp.float32),
    )(x, sw1, sw3, sw2)


def _dispatch_body(i0_hbm, i1_hbm, w0_hbm, w1_hbm,
                   gidx_hbm, slot0_hbm, slot1_hbm, wslot_hbm, te_hbm, act_hbm,
                   ev_ref, wv_ref, slots_ref, gidx_ref, wsl_ref, te_ref,
                   act_ref, sem):
    wid = lax.axis_index("s") * 2 + lax.axis_index("c")

    @pl.when(wid == 0)
    def _():
        pltpu.sync_copy(i0_hbm, ev_ref.at[pl.ds(0, T)])
        pltpu.sync_copy(i1_hbm, ev_ref.at[pl.ds(T, T)])
        pltpu.sync_copy(w0_hbm, wv_ref.at[pl.ds(0, T)])
        pltpu.sync_copy(w1_hbm, wv_ref.at[pl.ds(T, T)])

        nv = (2 * T) // L

        def c_body(i, cnts):
            v = ev_ref[pl.ds(i * L, L)]
            return tuple(cnts[e] + jnp.sum(jnp.where(v == e, 1, 0))
                         for e in range(E))

        cnts = lax.fori_loop(0, nv, c_body, (jnp.int32(0),) * E)

        base, nt_cum = [], []
        off = jnp.int32(0)
        tcum = jnp.int32(0)
        for e in range(E):
            p = ((cnts[e] + G - 1) // G) * G
            base.append(off)
            off = off + p
            tcum = tcum + p // G
            nt_cum.append(tcum)

        def z_body(i, _):
            lane = lax.iota(jnp.int32, L)
            gidx_ref[pl.ds(i * L, L)] = lax.bitwise_and(i * L + lane, T - 1)
            wsl_ref[pl.ds(i * L, L)] = jnp.zeros((L,), jnp.float32)
            return 0

        lax.fori_loop(0, T_PAD // L, z_body, 0)

        def p2_body(i, cnts):
            v = ev_ref[pl.ds(i * L, L)]
            w = wv_ref[pl.ds(i * L, L)]
            lane = lax.iota(jnp.int32, L)
            tok = lax.bitwise_and(i * L + lane, T - 1)
            slot_v = jnp.zeros((L,), jnp.int32)
            new = []
            for e in range(E):
                m = v == e
                cs = plsc.cumsum(jnp.where(m, 1, 0))
                slot_v = jnp.where(m, base[e] + cnts[e] + cs - 1, slot_v)
                new.append(cnts[e] + cs[L - 1])
            slots_ref[pl.ds(i * L, L)] = slot_v
            plsc.store_scatter(gidx_ref, [slot_v], tok)
            plsc.store_scatter(wsl_ref, [slot_v], w)
            return tuple(new)

        lax.fori_loop(0, nv, p2_body, (jnp.int32(0),) * E)

        for r in range(2):
            lane = lax.iota(jnp.int32, L) + r * L
            tev = jnp.zeros((L,), jnp.int32)
            for e in range(E):
                tev = tev + jnp.where(lane >= nt_cum[e], 1, 0)
            te_ref[pl.ds(r * L, L)] = jnp.minimum(tev, E - 1)
            act_ref[pl.ds(r * L, L)] = jnp.where(lane < nt_cum[E - 1], 1, 0)

        pltpu.sync_copy(slots_ref.at[pl.ds(0, T)], slot0_hbm)
        pltpu.sync_copy(slots_ref.at[pl.ds(T, T)], slot1_hbm)
        pltpu.sync_copy(gidx_ref, gidx_hbm)
        pltpu.sync_copy(wsl_ref, wslot_hbm)
        pltpu.sync_copy(te_ref, te_hbm)
        pltpu.sync_copy(act_ref, act_hbm)


def _dispatch(i0, i1, w0, w1):
    f = pl.kernel(
        _dispatch_body,
        out_type=[
            jax.ShapeDtypeStruct((T_PAD,), jnp.int32),
            jax.ShapeDtypeStruct((T,), jnp.int32),
            jax.ShapeDtypeStruct((T,), jnp.int32),
            jax.ShapeDtypeStruct((T_PAD,), jnp.float32),
            jax.ShapeDtypeStruct((2 * L,), jnp.int32),
            jax.ShapeDtypeStruct((2 * L,), jnp.int32),
        ],
        mesh=plsc.VectorSubcoreMesh(**_SC_MESH),
        scratch_types=[
            pltpu.VMEM((2 * T,), jnp.int32),
            pltpu.VMEM((2 * T,), jnp.float32),
            pltpu.VMEM((2 * T,), jnp.int32),
            pltpu.VMEM((T_PAD,), jnp.int32),
            pltpu.VMEM((T_PAD,), jnp.float32),
            pltpu.VMEM((2 * L,), jnp.int32),
            pltpu.VMEM((2 * L,), jnp.int32),
            pltpu.SemaphoreType.DMA,
        ],
        compiler_params=pltpu.CompilerParams(needs_layout_passes=False),
    )
    return f(i0, i1, w0, w1)


_G_ROWS = T_PAD // 32
_G_CHUNK = 24
_G_N = _G_ROWS // _G_CHUNK
_G_STAGE = T // 16


def _gather_body(x_hbm, gidx_hbm, xg_hbm,
                 idx_ref, b0, b1, b2, b3, s0, s1, s2, s3, w0, w1, w2, w3):
    cid = lax.axis_index("c")
    sid = lax.axis_index("s")
    wid = sid * 2 + cid
    base = wid * _G_ROWS
    pltpu.sync_copy(gidx_hbm.at[pl.ds(base, _G_ROWS)], idx_ref)
    bufs = (b0, b1, b2, b3)
    sems = (s0, s1, s2, s3)
    wsems = (w0, w1, w2, w3)
    pend_g, pend_w = {}, {}
    for c in range(_G_N):
        if c >= 4:
            pend_w.pop(c - 4).wait()
        pend_g[c] = pltpu.async_copy(
            x_hbm.at[idx_ref.at[pl.ds(c * _G_CHUNK, _G_CHUNK)]],
            bufs[c % 4], sems[c % 4])
        if c >= 3:
            d = c - 3
            pend_g.pop(d).wait()
            pend_w[d] = pltpu.async_copy(
                bufs[d % 4], xg_hbm.at[pl.ds(base + d * _G_CHUNK, _G_CHUNK)],
                wsems[d % 4])
    for d in range(_G_N - 3, _G_N):
        pend_g.pop(d).wait()
        pend_w[d] = pltpu.async_copy(
            bufs[d % 4], xg_hbm.at[pl.ds(base + d * _G_CHUNK, _G_CHUNK)],
            wsems[d % 4])
    for d in pend_w:
        pend_w[d].wait()


def _gather(x2d, gidx):
    f = pl.kernel(
        _gather_body,
        out_type=jax.ShapeDtypeStruct((T_PAD, C), jnp.float32),
        mesh=plsc.VectorSubcoreMesh(**_SC_MESH),
        scratch_types=[
            pltpu.VMEM((_G_ROWS,), jnp.int32),
            pltpu.VMEM((_G_CHUNK, C), jnp.float32),
            pltpu.VMEM((_G_CHUNK, C), jnp.float32),
            pltpu.VMEM((_G_CHUNK, C), jnp.float32),
            pltpu.VMEM((_G_CHUNK, C), jnp.float32),
            pltpu.SemaphoreType.DMA,
            pltpu.SemaphoreType.DMA,
            pltpu.SemaphoreType.DMA,
            pltpu.SemaphoreType.DMA,
            pltpu.SemaphoreType.DMA,
            pltpu.SemaphoreType.DMA,
            pltpu.SemaphoreType.DMA,
            pltpu.SemaphoreType.DMA,
        ],
    )
    return f(x2d, gidx)


def _grouped_body(te_ref, act_ref, xg_ref, w1_ref, w3_ref, w2_ref, ws_ref,
                  out_ref):
    @pl.when(act_ref[pl.program_id(0)] > 0)
    def _():
        xg = xg_ref[...]
        h = lax.dot_general(xg, w1_ref[0], (((1,), (1,)), ((), ())),
                            preferred_element_type=jnp.float32)
        g = lax.dot_general(xg, w3_ref[0], (((1,), (1,)), ((), ())),
                            preferred_element_type=jnp.float32)
        u = (h * jax.nn.sigmoid(h)) * g
        y = lax.dot_general(u, w2_ref[0], (((1,), (1,)), ((), ())),
                            preferred_element_type=jnp.float32)
        out_ref[...] = y * ws_ref[...]


def _grouped(xg, ew1, ew3, ew2, wslot2d, tile_expert, tile_act):
    grid_spec = pltpu.PrefetchScalarGridSpec(
        num_scalar_prefetch=2,
        grid=(N_TILES,),
        in_specs=[
            pl.BlockSpec((G, C), lambda i, te, act: (i, 0)),
            pl.BlockSpec((1, H, C), lambda i, te, act: (te[i], 0, 0)),
            pl.BlockSpec((1, H, C), lambda i, te, act: (te[i], 0, 0)),
            pl.BlockSpec((1, C, H), lambda i, te, act: (te[i], 0, 0)),
            pl.BlockSpec((G, 1), lambda i, te, act: (i, 0)),
        ],
        out_specs=pl.BlockSpec((G, C), lambda i, te, act: (i, 0)),
    )
    return pl.pallas_call(
        _grouped_body,
        grid_spec=grid_spec,
        out_shape=jax.ShapeDtypeStruct((T_PAD, C), jnp.float32),
    )(tile_expert, tile_act, xg, ew1, ew3, ew2, wslot2d)


_C_TOK = 64
_C_CHUNK = 8
_C_N = _C_TOK // _C_CHUNK


def _combine_body(sh_hbm, yg_hbm, slot0_hbm, slot1_hbm, out_hbm,
                  idx0_ref, idx1_ref, *rest):
    accs = rest[0:4]
    r0s = rest[4:10]
    r1s = rest[10:16]
    shsems = rest[16:20]
    osems = rest[20:24]
    sems = rest[24:30]
    wid = lax.axis_index("s") * 2 + lax.axis_index("c")
    base = wid * _C_TOK
    pltpu.sync_copy(slot0_hbm.at[pl.ds(base, _C_TOK)], idx0_ref)
    pltpu.sync_copy(slot1_hbm.at[pl.ds(base, _C_TOK)], idx1_ref)
    pending = {}
    pend_w = {}

    def issue(c):
        s = pl.ds(c * _C_CHUNK, _C_CHUNK)
        a = c % 4
        if c >= 4:
            pend_w.pop(c - 4).wait()
        h0 = pltpu.async_copy(yg_hbm.at[idx0_ref.at[s]], r0s[c % 6],
                              sems[c % 6])
        h1 = pltpu.async_copy(yg_hbm.at[idx1_ref.at[s]], r1s[c % 6],
                              sems[c % 6])
        hs = pltpu.async_copy(sh_hbm.at[pl.ds(base + c * _C_CHUNK, _C_CHUNK)],
                              accs[a], shsems[a])
        pending[c] = (h0, h1, hs)

    def finish(c):
        b = c % 6
        a = c % 4
        h0, h1, hs = pending.pop(c)
        o = base + c * _C_CHUNK
        hs.wait()
        h0.wait()
        h1.wait()

        def add_body(j, _):
            t = j // (C // L)
            k = (j % (C // L)) * L
            accs[a][t, pl.ds(k, L)] = (accs[a][t, pl.ds(k, L)]
                                       + r0s[b][t, pl.ds(k, L)]
                                       + r1s[b][t, pl.ds(k, L)])
            return 0

        lax.fori_loop(0, _C_CHUNK * (C // L), add_body, 0)
        pend_w[c] = pltpu.async_copy(accs[a], out_hbm.at[pl.ds(o, _C_CHUNK)],
                                     osems[a])

    for c in range(_C_N):
        if c >= 3:
            finish(c - 3)
        issue(c)
    for d in range(_C_N - 3, _C_N):
        finish(d)
    for d in pend_w:
        pend_w[d].wait()


def _combine(sh, yg, slot0, slot1):
    f = pl.kernel(
        _combine_body,
        out_type=jax.ShapeDtypeStruct((T, C), jnp.float32),
        mesh=plsc.VectorSubcoreMesh(**_SC_MESH),
        scratch_types=(
            [pltpu.VMEM((_C_TOK,), jnp.int32)] * 2
            + [pltpu.VMEM((_C_CHUNK, C), jnp.float32)] * 16
            + [pltpu.SemaphoreType.DMA] * 14
        ),
    )
    return f(sh, yg, slot0, slot1)


def kernel(x, gate_w, sw1, sw3, sw2, ew1, ew3, ew2):
    x2d = x.reshape(T, C)
    logits, i0, i1, w0, w1 = _router(x2d, gate_w)
    gidx, slot0, slot1, wslot, te, act = _dispatch(
        i0.reshape(T), i1.reshape(T), w0.reshape(T), w1.reshape(T))
    sh = _shared(x2d, sw1, sw3, sw2)
    xg = _gather(x2d, gidx)
    yg = _grouped(xg, ew1, ew3, ew2, wslot.reshape(T_PAD, 1), te, act)
    out = _combine(sh, yg, slot0, slot1)
    return out.reshape(x.shape), logits.reshape(x.shape[0], T, E)


# speedup vs baseline: 1.0438x; 1.0014x over previous
import jax
import jax.numpy as jnp
from jax import lax
from jax.experimental import pallas as pl
from jax.experimental.pallas import tpu as pltpu
from jax.experimental.pallas import tpu_sc as plsc

T, C, H, E, K = 2048, 768, 2048, 8, 2
G = 256
N_TILES = (T * K + E * (G - 1) + G - 1) // G
T_PAD = N_TILES * G
TG = 256
L = 16

_SC_MESH = dict(core_axis_name="c", subcore_axis_name="s", num_cores=2,
                num_subcores=16)


def _router_body(x_ref, gw_ref, logits_ref, i0_ref, i1_ref, w0_ref, w1o_ref):
    x = x_ref[...]
    logits = lax.dot_general(x, gw_ref[...], (((1,), (1,)), ((), ())),
                             preferred_element_type=jnp.float32)
    logits_ref[...] = logits
    idx = lax.broadcasted_iota(jnp.int32, logits.shape, 1)
    m1 = jnp.max(logits, axis=1, keepdims=True)
    i0 = jnp.min(jnp.where(logits == m1, idx, E), axis=1, keepdims=True)
    masked = jnp.where(idx == i0, -jnp.inf, logits)
    m2 = jnp.max(masked, axis=1, keepdims=True)
    i1 = jnp.min(jnp.where(masked == m2, idx, E), axis=1, keepdims=True)
    d = jnp.exp(m2 - m1)
    i0_ref[...] = i0
    i1_ref[...] = i1
    w0_ref[...] = 1.0 / (1.0 + d)
    w1o_ref[...] = d / (1.0 + d)


def _router(x, gate_w):
    return pl.pallas_call(
        _router_body,
        out_shape=[
            jax.ShapeDtypeStruct((T, E), jnp.float32),
            jax.ShapeDtypeStruct((T, 1), jnp.int32),
            jax.ShapeDtypeStruct((T, 1), jnp.int32),
            jax.ShapeDtypeStruct((T, 1), jnp.float32),
            jax.ShapeDtypeStruct((T, 1), jnp.float32),
        ],
    )(x, gate_w)


def _shared_body(x_ref, w1_ref, w3_ref, w2_ref, sh_ref):
    x = x_ref[...]
    h = lax.dot_general(x, w1_ref[...], (((1,), (1,)), ((), ())),
                        preferred_element_type=jnp.float32)
    g = lax.dot_general(x, w3_ref[...], (((1,), (1,)), ((), ())),
                        preferred_element_type=jnp.float32)
    u = (h * jax.nn.sigmoid(h)) * g
    sh_ref[...] = lax.dot_general(u, w2_ref[...], (((1,), (1,)), ((), ())),
                                  preferred_element_type=jnp.float32)


def _shared(x, sw1, sw3, sw2):
    n = T // TG
    return pl.pallas_call(
        _shared_body,
        grid=(n,),
        in_specs=[
            pl.BlockSpec((TG, C), lambda i: (i, 0)),
            pl.BlockSpec((H, C), lambda i: (0, 0)),
            pl.BlockSpec((H, C), lambda i: (0, 0)),
            pl.BlockSpec((C, H), lambda i: (0, 0)),
        ],
        out_specs=pl.BlockSpec((TG, C), lambda i: (i, 0)),
        out_shape=jax.ShapeDtypeStruct((T, C), jnp.float32),
    )(x, sw1, sw3, sw2)


def _dispatch_body(i0_hbm, i1_hbm, w0_hbm, w1_hbm,
                   gidx_hbm, slot0_hbm, slot1_hbm, wslot_hbm, te_hbm, act_hbm,
                   ev_ref, wv_ref, slots_ref, gidx_ref, wsl_ref, te_ref,
                   act_ref, sem):
    wid = lax.axis_index("s") * 2 + lax.axis_index("c")

    @pl.when(wid == 0)
    def _():
        pltpu.sync_copy(i0_hbm, ev_ref.at[pl.ds(0, T)])
        pltpu.sync_copy(i1_hbm, ev_ref.at[pl.ds(T, T)])
        pltpu.sync_copy(w0_hbm, wv_ref.at[pl.ds(0, T)])
        pltpu.sync_copy(w1_hbm, wv_ref.at[pl.ds(T, T)])

        nv = (2 * T) // L

        def c_body(i, cnts):
            v = ev_ref[pl.ds(i * L, L)]
            return tuple(cnts[e] + jnp.sum(jnp.where(v == e, 1, 0))
                         for e in range(E))

        cnts = lax.fori_loop(0, nv, c_body, (jnp.int32(0),) * E)

        base, nt_cum = [], []
        off = jnp.int32(0)
        tcum = jnp.int32(0)
        for e in range(E):
            p = ((cnts[e] + G - 1) // G) * G
            base.append(off)
            off = off + p
            tcum = tcum + p // G
            nt_cum.append(tcum)

        def z_body(i, _):
            lane = lax.iota(jnp.int32, L)
            gidx_ref[pl.ds(i * L, L)] = lax.bitwise_and(i * L + lane, T - 1)
            wsl_ref[pl.ds(i * L, L)] = jnp.zeros((L,), jnp.float32)
            return 0

        lax.fori_loop(0, T_PAD // L, z_body, 0)

        def p2_body(i, cnts):
            v = ev_ref[pl.ds(i * L, L)]
            w = wv_ref[pl.ds(i * L, L)]
            lane = lax.iota(jnp.int32, L)
            tok = lax.bitwise_and(i * L + lane, T - 1)
            slot_v = jnp.zeros((L,), jnp.int32)
            new = []
            for e in range(E):
                m = v == e
                cs = plsc.cumsum(jnp.where(m, 1, 0))
                slot_v = jnp.where(m, base[e] + cnts[e] + cs - 1, slot_v)
                new.append(cnts[e] + cs[L - 1])
            slots_ref[pl.ds(i * L, L)] = slot_v
            plsc.store_scatter(gidx_ref, [slot_v], tok)
            plsc.store_scatter(wsl_ref, [slot_v], w)
            return tuple(new)

        lax.fori_loop(0, nv, p2_body, (jnp.int32(0),) * E)

        for r in range(2):
            lane = lax.iota(jnp.int32, L) + r * L
            tev = jnp.zeros((L,), jnp.int32)
            for e in range(E):
                tev = tev + jnp.where(lane >= nt_cum[e], 1, 0)
            te_ref[pl.ds(r * L, L)] = jnp.minimum(tev, E - 1)
            act_ref[pl.ds(r * L, L)] = jnp.where(lane < nt_cum[E - 1], 1, 0)

        pltpu.sync_copy(slots_ref.at[pl.ds(0, T)], slot0_hbm)
        pltpu.sync_copy(slots_ref.at[pl.ds(T, T)], slot1_hbm)
        pltpu.sync_copy(gidx_ref, gidx_hbm)
        pltpu.sync_copy(wsl_ref, wslot_hbm)
        pltpu.sync_copy(te_ref, te_hbm)
        pltpu.sync_copy(act_ref, act_hbm)


def _dispatch(i0, i1, w0, w1):
    f = pl.kernel(
        _dispatch_body,
        out_type=[
            jax.ShapeDtypeStruct((T_PAD,), jnp.int32),
            jax.ShapeDtypeStruct((T,), jnp.int32),
            jax.ShapeDtypeStruct((T,), jnp.int32),
            jax.ShapeDtypeStruct((T_PAD,), jnp.float32),
            jax.ShapeDtypeStruct((2 * L,), jnp.int32),
            jax.ShapeDtypeStruct((2 * L,), jnp.int32),
        ],
        mesh=plsc.VectorSubcoreMesh(**_SC_MESH),
        scratch_types=[
            pltpu.VMEM((2 * T,), jnp.int32),
            pltpu.VMEM((2 * T,), jnp.float32),
            pltpu.VMEM((2 * T,), jnp.int32),
            pltpu.VMEM((T_PAD,), jnp.int32),
            pltpu.VMEM((T_PAD,), jnp.float32),
            pltpu.VMEM((2 * L,), jnp.int32),
            pltpu.VMEM((2 * L,), jnp.int32),
            pltpu.SemaphoreType.DMA,
        ],
        compiler_params=pltpu.CompilerParams(needs_layout_passes=False),
    )
    return f(i0, i1, w0, w1)


_G_ROWS = T_PAD // 32
_G_CHUNK = 32
_G_N = _G_ROWS // _G_CHUNK
_G_STAGE = T // 16


def _gather_body(x_hbm, gidx_hbm, xg_hbm,
                 idx_ref, b0, b1, b2, b3, s0, s1, s2, s3, w0, w1, w2, w3):
    cid = lax.axis_index("c")
    sid = lax.axis_index("s")
    wid = sid * 2 + cid
    base = wid * _G_ROWS
    pltpu.sync_copy(gidx_hbm.at[pl.ds(base, _G_ROWS)], idx_ref)
    bufs = (b0, b1, b2, b3)
    sems = (s0, s1, s2, s3)
    wsems = (w0, w1, w2, w3)
    pend_g, pend_w = {}, {}
    for c in range(_G_N):
        if c >= 4:
            pend_w.pop(c - 4).wait()
        pend_g[c] = pltpu.async_copy(
            x_hbm.at[idx_ref.at[pl.ds(c * _G_CHUNK, _G_CHUNK)]],
            bufs[c % 4], sems[c % 4])
        if c >= 3:
            d = c - 3
            pend_g.pop(d).wait()
            pend_w[d] = pltpu.async_copy(
                bufs[d % 4], xg_hbm.at[pl.ds(base + d * _G_CHUNK, _G_CHUNK)],
                wsems[d % 4])
    for d in range(_G_N - 3, _G_N):
        pend_g.pop(d).wait()
        pend_w[d] = pltpu.async_copy(
            bufs[d % 4], xg_hbm.at[pl.ds(base + d * _G_CHUNK, _G_CHUNK)],
            wsems[d % 4])
    for d in pend_w:
        pend_w[d].wait()


def _gather(x2d, gidx):
    f = pl.kernel(
        _gather_body,
        out_type=jax.ShapeDtypeStruct((T_PAD, C), jnp.float32),
        mesh=plsc.VectorSubcoreMesh(**_SC_MESH),
        scratch_types=[
            pltpu.VMEM((_G_ROWS,), jnp.int32),
            pltpu.VMEM((_G_CHUNK, C), jnp.float32),
            pltpu.VMEM((_G_CHUNK, C), jnp.float32),
            pltpu.VMEM((_G_CHUNK, C), jnp.float32),
            pltpu.VMEM((_G_CHUNK, C), jnp.float32),
            pltpu.SemaphoreType.DMA,
            pltpu.SemaphoreType.DMA,
            pltpu.SemaphoreType.DMA,
            pltpu.SemaphoreType.DMA,
            pltpu.SemaphoreType.DMA,
            pltpu.SemaphoreType.DMA,
            pltpu.SemaphoreType.DMA,
            pltpu.SemaphoreType.DMA,
        ],
    )
    return f(x2d, gidx)


def _grouped_body(te_ref, act_ref, xg_ref, w1_ref, w3_ref, w2_ref, ws_ref,
                  out_ref):
    @pl.when(act_ref[pl.program_id(0)] > 0)
    def _():
        xg = xg_ref[...]
        h = lax.dot_general(xg, w1_ref[0], (((1,), (1,)), ((), ())),
                            preferred_element_type=jnp.float32)
        g = lax.dot_general(xg, w3_ref[0], (((1,), (1,)), ((), ())),
                            preferred_element_type=jnp.float32)
        u = (h * jax.nn.sigmoid(h)) * g
        y = lax.dot_general(u, w2_ref[0], (((1,), (1,)), ((), ())),
                            preferred_element_type=jnp.float32)
        out_ref[...] = y * ws_ref[...]


def _grouped(xg, ew1, ew3, ew2, wslot2d, tile_expert, tile_act):
    grid_spec = pltpu.PrefetchScalarGridSpec(
        num_scalar_prefetch=2,
        grid=(N_TILES,),
        in_specs=[
            pl.BlockSpec((G, C), lambda i, te, act: (i, 0)),
            pl.BlockSpec((1, H, C), lambda i, te, act: (te[i], 0, 0)),
            pl.BlockSpec((1, H, C), lambda i, te, act: (te[i], 0, 0)),
            pl.BlockSpec((1, C, H), lambda i, te, act: (te[i], 0, 0)),
            pl.BlockSpec((G, 1), lambda i, te, act: (i, 0)),
        ],
        out_specs=pl.BlockSpec((G, C), lambda i, te, act: (i, 0)),
    )
    return pl.pallas_call(
        _grouped_body,
        grid_spec=grid_spec,
        out_shape=jax.ShapeDtypeStruct((T_PAD, C), jnp.float32),
    )(tile_expert, tile_act, xg, ew1, ew3, ew2, wslot2d)


_C_TOK = 64
_C_CHUNK = 8
_C_N = _C_TOK // _C_CHUNK


def _combine_body(sh_hbm, yg_hbm, slot0_hbm, slot1_hbm, out_hbm,
                  idx0_ref, idx1_ref, *rest):
    accs = rest[0:4]
    r0s = rest[4:10]
    r1s = rest[10:16]
    shsems = rest[16:20]
    osems = rest[20:24]
    sems = rest[24:30]
    wid = lax.axis_index("s") * 2 + lax.axis_index("c")
    base = wid * _C_TOK
    pltpu.sync_copy(slot0_hbm.at[pl.ds(base, _C_TOK)], idx0_ref)
    pltpu.sync_copy(slot1_hbm.at[pl.ds(base, _C_TOK)], idx1_ref)
    pending = {}
    pend_w = {}

    def issue(c):
        s = pl.ds(c * _C_CHUNK, _C_CHUNK)
        a = c % 4
        if c >= 4:
            pend_w.pop(c - 4).wait()
        h0 = pltpu.async_copy(yg_hbm.at[idx0_ref.at[s]], r0s[c % 6],
                              sems[c % 6])
        h1 = pltpu.async_copy(yg_hbm.at[idx1_ref.at[s]], r1s[c % 6],
                              sems[c % 6])
        hs = pltpu.async_copy(sh_hbm.at[pl.ds(base + c * _C_CHUNK, _C_CHUNK)],
                              accs[a], shsems[a])
        pending[c] = (h0, h1, hs)

    def finish(c):
        b = c % 6
        a = c % 4
        h0, h1, hs = pending.pop(c)
        o = base + c * _C_CHUNK
        hs.wait()
        h0.wait()
        h1.wait()

        def add_body(j, _):
            t = j // (C // L)
            k = (j % (C // L)) * L
            accs[a][t, pl.ds(k, L)] = (accs[a][t, pl.ds(k, L)]
                                       + r0s[b][t, pl.ds(k, L)]
                                       + r1s[b][t, pl.ds(k, L)])
            return 0

        lax.fori_loop(0, _C_CHUNK * (C // L), add_body, 0)
        pend_w[c] = pltpu.async_copy(accs[a], out_hbm.at[pl.ds(o, _C_CHUNK)],
                                     osems[a])

    for c in range(_C_N):
        if c >= 3:
            finish(c - 3)
        issue(c)
    for d in range(_C_N - 3, _C_N):
        finish(d)
    for d in pend_w:
        pend_w[d].wait()


def _combine(sh, yg, slot0, slot1):
    f = pl.kernel(
        _combine_body,
        out_type=jax.ShapeDtypeStruct((T, C), jnp.float32),
        mesh=plsc.VectorSubcoreMesh(**_SC_MESH),
        scratch_types=(
            [pltpu.VMEM((_C_TOK,), jnp.int32)] * 2
            + [pltpu.VMEM((_C_CHUNK, C), jnp.float32)] * 16
            + [pltpu.SemaphoreType.DMA] * 14
        ),
    )
    return f(sh, yg, slot0, slot1)


def kernel(x, gate_w, sw1, sw3, sw2, ew1, ew3, ew2):
    x2d = x.reshape(T, C)
    logits, i0, i1, w0, w1 = _router(x2d, gate_w)
    gidx, slot0, slot1, wslot, te, act = _dispatch(
        i0.reshape(T), i1.reshape(T), w0.reshape(T), w1.reshape(T))
    sh = _shared(x2d, sw1, sw3, sw2)
    xg = _gather(x2d, gidx)
    yg = _grouped(xg, ew1, ew3, ew2, wslot.reshape(T_PAD, 1), te, act)
    out = _combine(sh, yg, slot0, slot1)
    return out.reshape(x.shape), logits.reshape(x.shape[0], T, E)
